# Initial kernel scaffold; baseline (speedup 1.0000x reference)
import jax
import jax.numpy as jnp
from jax.experimental import pallas as pl


def kernel(x, sdf, edge_index, coarse_x, coarse_y, W_pre0, b_pre0, W_pre1, b_pre1, W_pre2, b_pre2, W_end0, b_end0, W_end1, b_end1, W_end2, b_end2):
    raise NotImplementedError("write your pallas kernel here")


# speedup vs baseline: 10.8349x; 10.8349x over previous
import functools

import jax
import jax.numpy as jnp
from jax import lax
from jax.experimental import pallas as pl
from jax.experimental.pallas import tpu as pltpu
from jax.experimental.pallas import tpu_sc as plsc

N_REAL = 10000
NP = 10240
E_REAL = 320000
NC_SC = 2
NS_SC = 16
NW = NC_SC * NS_SC
CHUNK = 128
CPT = -(-E_REAL // (NW * CHUNK))
EP = NW * CHUNK * CPT
ROWS_PER_TILE = NP // NS_SC
RB_PER_TILE = ROWS_PER_TILE // CHUNK

NCP = 2048
BR = 256


def _make_mp(width):
    mesh = plsc.VectorSubcoreMesh(core_axis_name="c", subcore_axis_name="s")

    @functools.partial(
        pl.kernel, mesh=mesh,
        out_type=jax.ShapeDtypeStruct((NC_SC, NP, width), jnp.float32),
        scratch_types=[
            pltpu.VMEM((CHUNK,), jnp.int32),
            pltpu.VMEM((CHUNK,), jnp.int32),
            pltpu.VMEM((CHUNK, width), jnp.float32),
            pltpu.VMEM_SHARED((NP, width), jnp.float32),
            pltpu.SemaphoreType.DMA,
        ],
    )
    def mp(table_hbm, src_hbm, dst_hbm, zeros_hbm, acc_out,
           sidx, didx, rows, acc, sem):
        c = lax.axis_index("c")
        s = lax.axis_index("s")
        wid = s * NC_SC + c
        pltpu.sync_copy(zeros_hbm, rows)
        for k in range(RB_PER_TILE):
            off = s * ROWS_PER_TILE + k * CHUNK
            pltpu.sync_copy(rows, acc.at[pl.ds(off, CHUNK)])
        plsc.subcore_barrier()
        base0 = wid * (CHUNK * CPT)

        def body(g, carry):
            b = pl.multiple_of(base0 + g * CHUNK, CHUNK)
            pltpu.sync_copy(dst_hbm.at[pl.ds(b, CHUNK)], didx)
            pltpu.sync_copy(src_hbm.at[pl.ds(b, CHUNK)], sidx)
            pltpu.async_copy(table_hbm.at[sidx], rows, sem).wait()
            pltpu.sync_copy(rows, acc.at[didx], add=True)
            return carry

        lax.fori_loop(0, CPT, body, 0)
        plsc.subcore_barrier()
        for k in range(RB_PER_TILE):
            off = s * ROWS_PER_TILE + k * CHUNK
            pltpu.sync_copy(acc.at[pl.ds(off, CHUNK)], rows)
            pltpu.sync_copy(rows, acc_out.at[c, pl.ds(off, CHUNK)])

    return mp


def _make_deg():
    mesh = plsc.VectorSubcoreMesh(core_axis_name="c", subcore_axis_name="s")

    @functools.partial(
        pl.kernel, mesh=mesh,
        out_type=jax.ShapeDtypeStruct((NC_SC, NP), jnp.float32),
        scratch_types=[
            pltpu.VMEM((CHUNK,), jnp.int32),
            pltpu.VMEM((CHUNK,), jnp.float32),
            pltpu.VMEM((ROWS_PER_TILE,), jnp.float32),
            pltpu.VMEM_SHARED((NP,), jnp.float32),
        ],
    )
    def deg(ones_hbm, dst_hbm, zeros_hbm, acc_out, didx, ones_v, zbuf, acc):
        c = lax.axis_index("c")
        s = lax.axis_index("s")
        wid = s * NC_SC + c
        pltpu.sync_copy(zeros_hbm, zbuf)
        pltpu.sync_copy(zbuf, acc.at[pl.ds(s * ROWS_PER_TILE, ROWS_PER_TILE)])
        plsc.subcore_barrier()
        pltpu.sync_copy(ones_hbm, ones_v)
        base0 = wid * (CHUNK * CPT)

        def body(g, carry):
            b = pl.multiple_of(base0 + g * CHUNK, CHUNK)
            pltpu.sync_copy(dst_hbm.at[pl.ds(b, CHUNK)], didx)
            pltpu.sync_copy(ones_v, acc.at[didx], add=True)
            return carry

        lax.fori_loop(0, CPT, body, 0)
        plsc.subcore_barrier()
        pltpu.sync_copy(acc.at[pl.ds(s * ROWS_PER_TILE, ROWS_PER_TILE)], zbuf)
        pltpu.sync_copy(zbuf, acc_out.at[c, pl.ds(s * ROWS_PER_TILE, ROWS_PER_TILE)])

    return deg


def _k_first(dacc_ref, fx_ref, w_ref, dinv_ref, hp_ref):
    deg = dacc_ref[0, :, 0:1] + dacc_ref[1, :, 0:1] + 1.0
    dinv = lax.rsqrt(deg)
    dinv_ref[...] = dinv
    h = jnp.dot(fx_ref[...], w_ref[...], preferred_element_type=jnp.float32)
    hp_ref[...] = h * dinv


def _k_mid(acc_ref, hp_ref, dinv_ref, b_ref, w_ref, out_ref):
    dinv = dinv_ref[...]
    pre = dinv * (acc_ref[0] + acc_ref[1] + hp_ref[...]) + b_ref[...]
    act = jnp.maximum(pre, 0.0)
    h = jnp.dot(act, w_ref[...], preferred_element_type=jnp.float32)
    out_ref[...] = h * dinv


def _k_mid_y(acc_ref, hp_ref, dinv_ref, b_ref, wy_ref, wx_ref, y3_ref, out_ref):
    dinv = dinv_ref[...]
    pre = dinv * (acc_ref[0] + acc_ref[1] + hp_ref[...]) + b_ref[...]
    act = jnp.maximum(pre, 0.0)
    h = (jnp.dot(y3_ref[...], wy_ref[...], preferred_element_type=jnp.float32)
         + jnp.dot(act, wx_ref[...], preferred_element_type=jnp.float32))
    out_ref[...] = h * dinv


def _k_final(acc_ref, hp_ref, dinv_ref, b_ref, out_ref):
    out_ref[...] = (dinv_ref[...] * (acc_ref[0] + acc_ref[1] + hp_ref[...])
                    + b_ref[...])


def _k_knn(pf_ref, pcT_ref, cy_ref, out_ref):
    pf = pf_ref[...]
    pcx = pcT_ref[0:1, :]
    pcy = pcT_ref[1:2, :]
    dx = pf[:, 0:1] - pcx
    dy = pf[:, 1:2] - pcy
    d2 = dx * dx + dy * dy
    cols = lax.broadcasted_iota(jnp.int32, (BR, NCP), 1)
    num = jnp.zeros((BR, 3), jnp.float32)
    wsum = jnp.zeros((BR, 1), jnp.float32)
    rem = d2
    for _ in range(3):
        m = jnp.min(rem, axis=1, keepdims=True)
        first = jnp.min(jnp.where(rem == m, cols, NCP), axis=1, keepdims=True)
        sel = jnp.where(cols == first, 1.0, 0.0)
        w = 1.0 / jnp.maximum(m, 1e-16)
        num = num + w * jnp.dot(sel, cy_ref[...],
                                preferred_element_type=jnp.float32)
        wsum = wsum + w
        rem = jnp.where(cols == first, jnp.float32(3e38), rem)
    out_ref[...] = num / wsum


def kernel(x, sdf, edge_index, coarse_x, coarse_y,
           W_pre0, b_pre0, W_pre1, b_pre1, W_pre2, b_pre2,
           W_end0, b_end0, W_end1, b_end1, W_end2, b_end2):
    f32 = jnp.float32
    xp = jnp.pad(x, ((0, NP - N_REAL), (0, 0)))
    sdfp = jnp.pad(sdf, ((0, NP - N_REAL), (0, 0)))
    fx = jnp.concatenate([xp, sdfp], axis=1)
    pad = EP - E_REAL
    padi = jnp.arange(pad, dtype=jnp.int32)
    srcp = jnp.concatenate([edge_index[0], padi % N_REAL])
    dstp = jnp.concatenate([edge_index[1], N_REAL + padi % (NP - N_REAL)])
    zeros128 = jnp.zeros((CHUNK, 128), f32)
    zeros1d = jnp.zeros((ROWS_PER_TILE,), f32)
    ones1d = jnp.ones((CHUNK,), f32)
    pcT = jnp.pad(coarse_x[:, :2].T, ((0, 0), (0, NCP - coarse_x.shape[0])),
                  constant_values=1e9)
    cyp = jnp.pad(coarse_y[:, :3], ((0, NCP - coarse_y.shape[0]), (0, 0)))
    w_end2p = jnp.pad(W_end2, ((0, 0), (0, 128 - W_end2.shape[1])))
    b_end2p = jnp.pad(b_end2, (0, 128 - b_end2.shape[0]))[None, :]
    b2 = lambda b: b[None, :]

    mp128 = _make_mp(128)

    dacc = _make_deg()(ones1d, dstp, zeros1d)[:, :, None]

    dinv, hp = pl.pallas_call(
        _k_first,
        out_shape=[jax.ShapeDtypeStruct((NP, 1), f32),
                   jax.ShapeDtypeStruct((NP, 128), f32)],
    )(dacc, fx, W_pre0)

    y3 = pl.pallas_call(
        _k_knn,
        grid=(NP // BR,),
        in_specs=[pl.BlockSpec((BR, 2), lambda i: (i, 0)),
                  pl.BlockSpec((2, NCP), lambda i: (0, 0)),
                  pl.BlockSpec((NCP, 3), lambda i: (0, 0))],
        out_specs=pl.BlockSpec((BR, 3), lambda i: (i, 0)),
        out_shape=jax.ShapeDtypeStruct((NP, 3), f32),
    )(xp[:, :2], pcT, cyp)

    mid = pl.pallas_call(
        _k_mid,
        out_shape=jax.ShapeDtypeStruct((NP, 128), f32),
    )

    acc = mp128(hp, srcp, dstp, zeros128)
    hp = mid(acc, hp, dinv, b2(b_pre0), W_pre1)
    acc = mp128(hp, srcp, dstp, zeros128)
    hp = mid(acc, hp, dinv, b2(b_pre1), W_pre2)
    acc = mp128(hp, srcp, dstp, zeros128)
    hp = pl.pallas_call(
        _k_mid_y,
        out_shape=jax.ShapeDtypeStruct((NP, 128), f32),
    )(acc, hp, dinv, b2(b_pre2), W_end0[:3], W_end0[3:], y3)
    acc = mp128(hp, srcp, dstp, zeros128)
    hp = mid(acc, hp, dinv, b2(b_end0), W_end1)
    acc = mp128(hp, srcp, dstp, zeros128)
    hp = mid(acc, hp, dinv, b2(b_end1), w_end2p)
    acc = mp128(hp, srcp, dstp, zeros128)
    out = pl.pallas_call(
        _k_final,
        out_shape=jax.ShapeDtypeStruct((NP, 128), f32),
    )(acc, hp, dinv, b_end2p)
    return out[:N_REAL, :3]


# speedup vs baseline: 14.8755x; 1.3729x over previous
import functools

import jax
import jax.numpy as jnp
from jax import lax
from jax.experimental import pallas as pl
from jax.experimental.pallas import tpu as pltpu
from jax.experimental.pallas import tpu_sc as plsc

N_REAL = 10000
NP = 10240
E_REAL = 320000
NC_SC = 2
NS_SC = 16
NW = NC_SC * NS_SC
CHUNK = 128
NBUF = 2
CPT = NBUF * (-(-E_REAL // (NW * CHUNK * NBUF)))
EP = NW * CHUNK * CPT
ROWS_PER_TILE = NP // NS_SC
RB_PER_TILE = ROWS_PER_TILE // CHUNK

NCP = 2048
BR = 256


def _make_mp(width):
    mesh = plsc.VectorSubcoreMesh(core_axis_name="c", subcore_axis_name="s")

    @functools.partial(
        pl.kernel, mesh=mesh,
        out_type=jax.ShapeDtypeStruct((NC_SC, NP, width), jnp.float32),
        scratch_types=[
            pltpu.VMEM((NBUF, CHUNK), jnp.int32),
            pltpu.VMEM((NBUF, CHUNK), jnp.int32),
            pltpu.VMEM((NBUF, CHUNK, width), jnp.float32),
            pltpu.VMEM_SHARED((NP, width), jnp.float32),
            pltpu.SemaphoreType.DMA,
            pltpu.SemaphoreType.DMA,
        ],
    )
    def mp(table_hbm, src_hbm, dst_hbm, zeros_hbm, acc_out,
           sidx, didx, rows, acc, sem0, sem1):
        sems = (sem0, sem1)
        c = lax.axis_index("c")
        s = lax.axis_index("s")
        wid = s * NC_SC + c
        base0 = wid * (CHUNK * CPT)

        def load_idx(g, b):
            off = pl.multiple_of(base0 + g * CHUNK, CHUNK)
            pltpu.sync_copy(src_hbm.at[pl.ds(off, CHUNK)], sidx.at[b])
            pltpu.sync_copy(dst_hbm.at[pl.ds(off, CHUNK)], didx.at[b])

        pltpu.sync_copy(zeros_hbm, rows.at[0])
        for k in range(RB_PER_TILE):
            off = s * ROWS_PER_TILE + k * CHUNK
            pltpu.sync_copy(rows.at[0], acc.at[pl.ds(off, CHUNK)])
        plsc.subcore_barrier()

        def outer(go, carry):
            g0 = go * NBUF
            handles = []
            for b in range(NBUF):
                load_idx(g0 + b, b)
                handles.append(
                    pltpu.async_copy(table_hbm.at[sidx.at[b]],
                                     rows.at[b], sems[b]))
            for b in range(NBUF):
                handles[b].wait()
                pltpu.sync_copy(rows.at[b], acc.at[didx.at[b]], add=True)
            return carry

        lax.fori_loop(0, CPT // NBUF, outer, 0)
        plsc.subcore_barrier()
        for k in range(RB_PER_TILE):
            off = s * ROWS_PER_TILE + k * CHUNK
            bb = k % NBUF
            pltpu.sync_copy(acc.at[pl.ds(off, CHUNK)], rows.at[bb])
            pltpu.sync_copy(rows.at[bb], acc_out.at[c, pl.ds(off, CHUNK)])

    return mp


def _make_deg():
    mesh = plsc.VectorSubcoreMesh(core_axis_name="c", subcore_axis_name="s")

    @functools.partial(
        pl.kernel, mesh=mesh,
        out_type=jax.ShapeDtypeStruct((NC_SC, NP), jnp.float32),
        scratch_types=[
            pltpu.VMEM((CPT, CHUNK), jnp.int32),
            pltpu.VMEM((CHUNK,), jnp.float32),
            pltpu.VMEM((ROWS_PER_TILE,), jnp.float32),
            pltpu.VMEM_SHARED((NP,), jnp.float32),
        ],
    )
    def deg(ones_hbm, dst_hbm, zeros_hbm, acc_out, didx, ones_v, zbuf, acc):
        c = lax.axis_index("c")
        s = lax.axis_index("s")
        wid = s * NC_SC + c
        pltpu.sync_copy(dst_hbm.at[wid], didx)
        pltpu.sync_copy(zeros_hbm, zbuf)
        pltpu.sync_copy(zbuf, acc.at[pl.ds(s * ROWS_PER_TILE, ROWS_PER_TILE)])
        plsc.subcore_barrier()
        pltpu.sync_copy(ones_hbm, ones_v)

        def body(g, carry):
            pltpu.sync_copy(ones_v, acc.at[didx.at[g]], add=True)
            return carry

        lax.fori_loop(0, CPT, body, 0)
        plsc.subcore_barrier()
        pltpu.sync_copy(acc.at[pl.ds(s * ROWS_PER_TILE, ROWS_PER_TILE)], zbuf)
        pltpu.sync_copy(zbuf, acc_out.at[c, pl.ds(s * ROWS_PER_TILE, ROWS_PER_TILE)])

    return deg


def _k_first(dacc_ref, fx_ref, w_ref, dinv_ref, hp_ref):
    deg = dacc_ref[0, :, 0:1] + dacc_ref[1, :, 0:1] + 1.0
    dinv = lax.rsqrt(deg)
    dinv_ref[...] = dinv
    h = jnp.dot(fx_ref[...], w_ref[...], preferred_element_type=jnp.float32)
    hp_ref[...] = h * dinv


def _k_mid(acc_ref, hp_ref, dinv_ref, b_ref, w_ref, out_ref):
    dinv = dinv_ref[...]
    pre = dinv * (acc_ref[0] + acc_ref[1] + hp_ref[...]) + b_ref[...]
    act = jnp.maximum(pre, 0.0)
    h = jnp.dot(act, w_ref[...], preferred_element_type=jnp.float32)
    out_ref[...] = h * dinv


def _k_mid_y(acc_ref, hp_ref, dinv_ref, b_ref, wy_ref, wx_ref, y3_ref, out_ref):
    dinv = dinv_ref[...]
    pre = dinv * (acc_ref[0] + acc_ref[1] + hp_ref[...]) + b_ref[...]
    act = jnp.maximum(pre, 0.0)
    h = (jnp.dot(y3_ref[...], wy_ref[...], preferred_element_type=jnp.float32)
         + jnp.dot(act, wx_ref[...], preferred_element_type=jnp.float32))
    out_ref[...] = h * dinv


def _k_final(acc_ref, hp_ref, dinv_ref, b_ref, out_ref):
    out_ref[...] = (dinv_ref[...] * (acc_ref[0] + acc_ref[1] + hp_ref[...])
                    + b_ref[...])


def _k_knn(pf_ref, pcT_ref, cy_ref, out_ref):
    pf = pf_ref[...]
    pcx = pcT_ref[0:1, :]
    pcy = pcT_ref[1:2, :]
    dx = pf[:, 0:1] - pcx
    dy = pf[:, 1:2] - pcy
    d2 = dx * dx + dy * dy
    cols = lax.broadcasted_iota(jnp.int32, (BR, NCP), 1)
    num = jnp.zeros((BR, 3), jnp.float32)
    wsum = jnp.zeros((BR, 1), jnp.float32)
    rem = d2
    for _ in range(3):
        m = jnp.min(rem, axis=1, keepdims=True)
        first = jnp.min(jnp.where(rem == m, cols, NCP), axis=1, keepdims=True)
        sel = jnp.where(cols == first, 1.0, 0.0)
        w = 1.0 / jnp.maximum(m, 1e-16)
        num = num + w * jnp.dot(sel, cy_ref[...],
                                preferred_element_type=jnp.float32)
        wsum = wsum + w
        rem = jnp.where(cols == first, jnp.float32(3e38), rem)
    out_ref[...] = num / wsum


def kernel(x, sdf, edge_index, coarse_x, coarse_y,
           W_pre0, b_pre0, W_pre1, b_pre1, W_pre2, b_pre2,
           W_end0, b_end0, W_end1, b_end1, W_end2, b_end2):
    f32 = jnp.float32
    xp = jnp.pad(x, ((0, NP - N_REAL), (0, 0)))
    sdfp = jnp.pad(sdf, ((0, NP - N_REAL), (0, 0)))
    fx = jnp.concatenate([xp, sdfp], axis=1)
    pad = EP - E_REAL
    padi = jnp.arange(pad, dtype=jnp.int32)
    srcp = jnp.concatenate([edge_index[0], padi % N_REAL])
    dstp = jnp.concatenate([edge_index[1], N_REAL + padi % (NP - N_REAL)])
    src3 = srcp.reshape(NW, CPT, CHUNK)
    dst3 = dstp.reshape(NW, CPT, CHUNK)
    zeros128 = jnp.zeros((CHUNK, 128), f32)
    zeros1d = jnp.zeros((ROWS_PER_TILE,), f32)
    ones1d = jnp.ones((CHUNK,), f32)
    pcT = jnp.pad(coarse_x[:, :2].T, ((0, 0), (0, NCP - coarse_x.shape[0])),
                  constant_values=1e9)
    cyp = jnp.pad(coarse_y[:, :3], ((0, NCP - coarse_y.shape[0]), (0, 0)))
    w_end2p = jnp.pad(W_end2, ((0, 0), (0, 128 - W_end2.shape[1])))
    b_end2p = jnp.pad(b_end2, (0, 128 - b_end2.shape[0]))[None, :]
    b2 = lambda b: b[None, :]

    mp128 = _make_mp(128)

    dacc = _make_deg()(ones1d, dst3, zeros1d)[:, :, None]

    dinv, hp = pl.pallas_call(
        _k_first,
        out_shape=[jax.ShapeDtypeStruct((NP, 1), f32),
                   jax.ShapeDtypeStruct((NP, 128), f32)],
    )(dacc, fx, W_pre0)

    y3 = pl.pallas_call(
        _k_knn,
        grid=(NP // BR,),
        in_specs=[pl.BlockSpec((BR, 2), lambda i: (i, 0)),
                  pl.BlockSpec((2, NCP), lambda i: (0, 0)),
                  pl.BlockSpec((NCP, 3), lambda i: (0, 0))],
        out_specs=pl.BlockSpec((BR, 3), lambda i: (i, 0)),
        out_shape=jax.ShapeDtypeStruct((NP, 3), f32),
    )(xp[:, :2], pcT, cyp)

    mid = pl.pallas_call(
        _k_mid,
        out_shape=jax.ShapeDtypeStruct((NP, 128), f32),
    )

    acc = mp128(hp, srcp, dstp, zeros128)
    hp = mid(acc, hp, dinv, b2(b_pre0), W_pre1)
    acc = mp128(hp, srcp, dstp, zeros128)
    hp = mid(acc, hp, dinv, b2(b_pre1), W_pre2)
    acc = mp128(hp, srcp, dstp, zeros128)
    hp = pl.pallas_call(
        _k_mid_y,
        out_shape=jax.ShapeDtypeStruct((NP, 128), f32),
    )(acc, hp, dinv, b2(b_pre2), W_end0[:3], W_end0[3:], y3)
    acc = mp128(hp, srcp, dstp, zeros128)
    hp = mid(acc, hp, dinv, b2(b_end0), W_end1)
    acc = mp128(hp, srcp, dstp, zeros128)
    hp = mid(acc, hp, dinv, b2(b_end1), w_end2p)
    acc = mp128(hp, srcp, dstp, zeros128)
    out = pl.pallas_call(
        _k_final,
        out_shape=jax.ShapeDtypeStruct((NP, 128), f32),
    )(acc, hp, dinv, b_end2p)
    return out[:N_REAL, :3]
